# baseline (speedup 1.0000x reference)
import numpy as np

import jax
import jax.numpy as jnp
from jax import lax
from jax.experimental import pallas as pl
from jax.experimental.pallas import tpu as pltpu
from jax.experimental.pallas import tpu_sc as plsc
from jax.experimental.layout import Format, Layout, with_layout_constraint

F = 26
B = 4096
EPS = 1e-5
NC, NS, L = 2, 16, 16
CH = 1024
NCH = B // CH
IDXROW = 128
GPC = CH // IDXROW

D8, D16, D32 = 8, 16, 32
COL16 = F * D8
COL32 = F * (D8 + D16)
OUT_D = F * (D8 + D16 + D32)
PAD_D = 1536
GROWS = B * PAD_D // 128
NBB = 16
BB = B // NBB
GBB = BB * PAD_D // 128
NF = 96

_M = np.zeros((PAD_D, NF), np.float32)
_DEN = np.ones((1, NF), np.float32)
_MASK = np.zeros((1, PAD_D), np.float32)
_MASK[0, :OUT_D] = 1.0
for _f in range(F):
    _M[_f * D8 : (_f + 1) * D8, _f] = 1.0
    _DEN[0, _f] = B * D8
    _M[COL16 + _f * D16 : COL16 + (_f + 1) * D16, 32 + _f] = 1.0
    _DEN[0, 32 + _f] = B * D16
    _M[COL32 + _f * D32 : COL32 + (_f + 1) * D32, 64 + _f] = 1.0
    _DEN[0, 64 + _f] = B * D32


def _sc_body(ids3, t8, t16, t32, graw, idxv, b8, b16, b32, sem):
    wid = lax.axis_index("s") * NC + lax.axis_index("c")

    def unit(table, f, d, buf, col_base):
        pltpu.sync_copy(ids3.at[f], idxv)
        col = col_base + f * d

        def chunk(c, carry):
            hs = [
                pltpu.async_copy(
                    table.at[idxv.at[c * GPC + j]],
                    buf.at[pl.ds(j * IDXROW, IDXROW)],
                    sem,
                )
                for j in range(GPC)
            ]
            for h in hs:
                h.wait()
            pltpu.sync_copy(buf, graw.at[pl.ds(c * CH, CH), pl.ds(col, d)])
            return carry

        lax.fori_loop(0, NCH, chunk, 0)

    @pl.when(wid < F)
    def _():
        unit(t32, wid, D32, b32, COL32)
        unit(t16, wid, D16, b16, COL16)

    @pl.when(wid >= F)
    def _():
        def d8_slot(k, carry):
            f = (wid - F) + (NC * NS - F) * k

            @pl.when(f < F)
            def _():
                unit(t8, f, D8, b8, 0)

            return carry

        lax.fori_loop(0, 5, d8_slot, 0)


def _make_sc_gather():
    return pl.kernel(
        _sc_body,
        out_type=jax.ShapeDtypeStruct((B, PAD_D), jnp.float32),
        mesh=plsc.VectorSubcoreMesh(
            core_axis_name="c", subcore_axis_name="s", num_cores=NC, num_subcores=NS
        ),
        scratch_types=[
            pltpu.VMEM((B // IDXROW, IDXROW), jnp.int32),
            pltpu.VMEM((CH, D8), jnp.float32),
            pltpu.VMEM((CH, D16), jnp.float32),
            pltpu.VMEM((CH, D32), jnp.float32),
            pltpu.SemaphoreType.DMA,
        ],
        compiler_params=pltpu.CompilerParams(use_tc_tiling_on_sc=False),
    )


VB = 8192
VOCAB = 100000
NVB = -(-VOCAB // VB)


_E = {}
for _d in (D8, D16, D32):
    _g = 128 // _d
    _e = np.zeros((_g, _d, 128), np.float32)
    for _k in range(_g):
        for _c in range(_d):
            _e[_k, _c, _k * _d + _c] = 1.0
    _E[_d] = _e


def _lin_body(x8, x16, x32, e8, e16, e32, y8, y16, y32):
    for x, y, e, d in ((x8, y8, e8, D8), (x16, y16, e16, D16), (x32, y32, e32, D32)):
        g = 128 // d
        xt = x[...].T
        xh = xt.astype(jnp.bfloat16)
        xl = (xt - xh.astype(jnp.float32)).astype(jnp.bfloat16)
        eb = e[...].astype(jnp.bfloat16)
        acc = jnp.zeros((VB // g, 128), jnp.float32)
        for part in (xh, xl):
            xg = jnp.reshape(part, (VB // g, g, d))
            for k in range(g):
                acc = acc + jnp.dot(
                    xg[:, k, :], eb[k], preferred_element_type=jnp.float32
                )
        y[...] = acc


def _tc_linearize(e0t, e1t, e2t):
    return pl.pallas_call(
        _lin_body,
        grid=(NVB,),
        in_specs=[
            pl.BlockSpec((D8, VB), lambda i: (0, i)),
            pl.BlockSpec((D16, VB), lambda i: (0, i)),
            pl.BlockSpec((D32, VB), lambda i: (0, i)),
            pl.BlockSpec((128 // D8, D8, 128), lambda i: (0, 0, 0)),
            pl.BlockSpec((128 // D16, D16, 128), lambda i: (0, 0, 0)),
            pl.BlockSpec((128 // D32, D32, 128), lambda i: (0, 0, 0)),
        ],
        out_specs=[
            pl.BlockSpec((VB * D8 // 128, 128), lambda i: (i, 0)),
            pl.BlockSpec((VB * D16 // 128, 128), lambda i: (i, 0)),
            pl.BlockSpec((VB * D32 // 128, 128), lambda i: (i, 0)),
        ],
        out_shape=[
            jax.ShapeDtypeStruct((VOCAB * D8 // 128, 128), jnp.float32),
            jax.ShapeDtypeStruct((VOCAB * D16 // 128, 128), jnp.float32),
            jax.ShapeDtypeStruct((VOCAB * D32 // 128, 128), jnp.float32),
        ],
    )(
        e0t, e1t, e2t,
        jnp.asarray(_E[D8]), jnp.asarray(_E[D16]), jnp.asarray(_E[D32]),
    )


def _stats_body(mask_ref, g_ref, out):
    @pl.when(pl.program_id(0) == 0)
    def _():
        out[...] = jnp.zeros_like(out)

    x = jnp.reshape(g_ref[...], (BB, PAD_D))
    x = jnp.where(mask_ref[...] != 0.0, x, 0.0)
    out[0:1, :] += jnp.sum(x, axis=0, keepdims=True)
    out[1:2, :] += jnp.sum(x * x, axis=0, keepdims=True)


def _tc_stats(mask, graw1):
    return pl.pallas_call(
        _stats_body,
        grid=(NBB,),
        in_specs=[
            pl.BlockSpec((1, PAD_D), lambda i: (0, 0)),
            pl.BlockSpec((GBB, 128), lambda i: (i, 0)),
        ],
        out_specs=pl.BlockSpec((8, PAD_D), lambda i: (0, 0)),
        out_shape=jax.ShapeDtypeStruct((8, PAD_D), jnp.float32),
    )(mask, graw1)


def _affine_body(stats_ref, m_ref, c_ref, den_ref, g_ref, out, sb_ref):
    @pl.when(pl.program_id(0) == 0)
    def _():
        m = m_ref[...]
        hi = lax.Precision.HIGHEST
        s = jnp.dot(stats_ref[0:1, :], m, precision=hi)
        q = jnp.dot(stats_ref[1:2, :], m, precision=hi)
        den = den_ref[...]
        mean = s / den
        var = q / den - mean * mean
        scale = c_ref[...] * lax.rsqrt(var + EPS)
        bias = -mean * scale
        sb_ref[0:1, :] = lax.dot_general(
            scale, m, (((1,), (1,)), ((), ())), precision=hi
        )
        sb_ref[1:2, :] = lax.dot_general(
            bias, m, (((1,), (1,)), ((), ())), precision=hi
        )

    x = jnp.reshape(g_ref[...], (BB, PAD_D))
    xn = x * sb_ref[0:1, :] + sb_ref[1:2, :]
    out[...] = xn[:, 0:OUT_D]


def _tc_affine(stats, c_all, graw1):
    return pl.pallas_call(
        _affine_body,
        grid=(NBB,),
        in_specs=[
            pl.BlockSpec((8, PAD_D), lambda i: (0, 0)),
            pl.BlockSpec((PAD_D, NF), lambda i: (0, 0)),
            pl.BlockSpec((1, NF), lambda i: (0, 0)),
            pl.BlockSpec((1, NF), lambda i: (0, 0)),
            pl.BlockSpec((GBB, 128), lambda i: (i, 0)),
        ],
        out_specs=pl.BlockSpec((BB, OUT_D), lambda i: (i, 0)),
        out_shape=jax.ShapeDtypeStruct((B, OUT_D), jnp.float32),
        scratch_shapes=[pltpu.VMEM((8, PAD_D), jnp.float32)],
    )(stats, jnp.asarray(_M), c_all, jnp.asarray(_DEN), graw1)


def _linear8():
    return Layout(major_to_minor=(0, 1), tiling=((8,),))


@jax.jit
def kernel(input_ids, emb_0, emb_1, emb_2, fw_0, fw_1, fw_2, region_weights_raw):
    rw = jax.nn.softmax(region_weights_raw, axis=0)
    c_all = jnp.zeros((1, NF), jnp.float32)
    c_all = c_all.at[0, 0:F].set(fw_0[:, 0] * rw[0, 0])
    c_all = c_all.at[0, 32 : 32 + F].set(fw_1[:, 0] * rw[1, 0])
    c_all = c_all.at[0, 64 : 64 + F].set(fw_2[:, 0] * rw[2, 0])
    ids3 = input_ids.astype(jnp.int32).T.reshape(F, B // IDXROW, IDXROW)
    lin8, lin16, lin32 = _tc_linearize(emb_0.T, emb_1.T, emb_2.T)
    t8 = lin8.reshape(VOCAB, D8)
    t16 = lin16.reshape(VOCAB, D16)
    t32 = lin32.reshape(VOCAB, D32)
    graw = _make_sc_gather()(ids3, t8, t16, t32)
    graw1 = graw.reshape(GROWS, 128)
    stats = _tc_stats(jnp.asarray(_MASK), graw1)
    return _tc_affine(stats, c_all, graw1)


# speedup vs baseline: 1.4720x; 1.4720x over previous
import numpy as np

import jax
import jax.numpy as jnp
from jax import lax
from jax.experimental import pallas as pl
from jax.experimental.pallas import tpu as pltpu
from jax.experimental.pallas import tpu_sc as plsc
from jax.experimental.layout import Format, Layout, with_layout_constraint

F = 26
B = 4096
EPS = 1e-5
NC, NS, L = 2, 16, 16
CH = 1024
NCH = B // CH
IDXROW = 128
GPC = CH // IDXROW

D8, D16, D32 = 8, 16, 32
COL16 = F * D8
COL32 = F * (D8 + D16)
OUT_D = F * (D8 + D16 + D32)
PAD_D = 1536
GROWS = B * PAD_D // 128
NBB = 16
BB = B // NBB
GBB = BB * PAD_D // 128
NF = 96

_M = np.zeros((PAD_D, NF), np.float32)
_DEN = np.ones((1, NF), np.float32)
_MASK = np.zeros((1, PAD_D), np.float32)
_MASK[0, :OUT_D] = 1.0
for _f in range(F):
    _M[_f * D8 : (_f + 1) * D8, _f] = 1.0
    _DEN[0, _f] = B * D8
    _M[COL16 + _f * D16 : COL16 + (_f + 1) * D16, 32 + _f] = 1.0
    _DEN[0, 32 + _f] = B * D16
    _M[COL32 + _f * D32 : COL32 + (_f + 1) * D32, 64 + _f] = 1.0
    _DEN[0, 64 + _f] = B * D32


def _sc_body(ids3, t8, t16, t32, graw, idxv, b8, b16, b32, sem):
    wid = lax.axis_index("s") * NC + lax.axis_index("c")

    def unit(table, f, d, buf, col_base):
        pltpu.sync_copy(ids3.at[f], idxv)
        col = col_base + f * d

        def chunk(c, carry):
            hs = [
                pltpu.async_copy(
                    table.at[idxv.at[c * GPC + j]],
                    buf.at[pl.ds(j * IDXROW, IDXROW)],
                    sem,
                )
                for j in range(GPC)
            ]
            for h in hs:
                h.wait()
            pltpu.sync_copy(buf, graw.at[pl.ds(c * CH, CH), pl.ds(col, d)])
            return carry

        lax.fori_loop(0, NCH, chunk, 0)

    @pl.when(wid < F)
    def _():
        unit(t32, wid, D32, b32, COL32)
        unit(t16, wid, D16, b16, COL16)

    @pl.when(wid >= F)
    def _():
        def d8_slot(k, carry):
            f = (wid - F) + (NC * NS - F) * k

            @pl.when(f < F)
            def _():
                unit(t8, f, D8, b8, 0)

            return carry

        lax.fori_loop(0, 5, d8_slot, 0)


def _make_sc_gather():
    return pl.kernel(
        _sc_body,
        out_type=jax.ShapeDtypeStruct((B, PAD_D), jnp.float32),
        mesh=plsc.VectorSubcoreMesh(
            core_axis_name="c", subcore_axis_name="s", num_cores=NC, num_subcores=NS
        ),
        scratch_types=[
            pltpu.VMEM((B // IDXROW, IDXROW), jnp.int32),
            pltpu.VMEM((CH, D8), jnp.float32),
            pltpu.VMEM((CH, D16), jnp.float32),
            pltpu.VMEM((CH, D32), jnp.float32),
            pltpu.SemaphoreType.DMA,
        ],
        compiler_params=pltpu.CompilerParams(use_tc_tiling_on_sc=False),
    )


VB = 8192
VOCAB = 100000
NVB = -(-VOCAB // VB)


_E = {}
for _d in (D8, D16, D32):
    _g = 128 // _d
    _e = np.zeros((_g, _d, 128), np.float32)
    for _k in range(_g):
        for _c in range(_d):
            _e[_k, _c, _k * _d + _c] = 1.0
    _E[_d] = _e


def _lin_body(x8, x16, x32, e8, e16, e32, y8, y16, y32):
    del e8, e16, e32
    for x, y, d in ((x8, y8, D8), (x16, y16, D16), (x32, y32, D32)):
        g = 128 // d
        xt = x[...].T
        xg = jnp.reshape(xt, (VB // g, g, d))
        for k in range(g):
            y[:, k * d : (k + 1) * d] = xg[:, k, :]


def _tc_linearize(e0t, e1t, e2t):
    return pl.pallas_call(
        _lin_body,
        grid=(NVB,),
        in_specs=[
            pl.BlockSpec((D8, VB), lambda i: (0, i)),
            pl.BlockSpec((D16, VB), lambda i: (0, i)),
            pl.BlockSpec((D32, VB), lambda i: (0, i)),
            pl.BlockSpec((128 // D8, D8, 128), lambda i: (0, 0, 0)),
            pl.BlockSpec((128 // D16, D16, 128), lambda i: (0, 0, 0)),
            pl.BlockSpec((128 // D32, D32, 128), lambda i: (0, 0, 0)),
        ],
        out_specs=[
            pl.BlockSpec((VB * D8 // 128, 128), lambda i: (i, 0)),
            pl.BlockSpec((VB * D16 // 128, 128), lambda i: (i, 0)),
            pl.BlockSpec((VB * D32 // 128, 128), lambda i: (i, 0)),
        ],
        out_shape=[
            jax.ShapeDtypeStruct((VOCAB * D8 // 128, 128), jnp.float32),
            jax.ShapeDtypeStruct((VOCAB * D16 // 128, 128), jnp.float32),
            jax.ShapeDtypeStruct((VOCAB * D32 // 128, 128), jnp.float32),
        ],
    )(
        e0t, e1t, e2t,
        jnp.asarray(_E[D8]), jnp.asarray(_E[D16]), jnp.asarray(_E[D32]),
    )


def _stats_body(mask_ref, g_ref, out):
    @pl.when(pl.program_id(0) == 0)
    def _():
        out[...] = jnp.zeros_like(out)

    x = jnp.reshape(g_ref[...], (BB, PAD_D))
    x = jnp.where(mask_ref[...] != 0.0, x, 0.0)
    out[0:1, :] += jnp.sum(x, axis=0, keepdims=True)
    out[1:2, :] += jnp.sum(x * x, axis=0, keepdims=True)


def _tc_stats(mask, graw1):
    return pl.pallas_call(
        _stats_body,
        grid=(NBB,),
        in_specs=[
            pl.BlockSpec((1, PAD_D), lambda i: (0, 0)),
            pl.BlockSpec((GBB, 128), lambda i: (i, 0)),
        ],
        out_specs=pl.BlockSpec((8, PAD_D), lambda i: (0, 0)),
        out_shape=jax.ShapeDtypeStruct((8, PAD_D), jnp.float32),
    )(mask, graw1)


def _affine_body(stats_ref, m_ref, c_ref, den_ref, g_ref, out, sb_ref):
    @pl.when(pl.program_id(0) == 0)
    def _():
        m = m_ref[...]
        hi = lax.Precision.HIGHEST
        s = jnp.dot(stats_ref[0:1, :], m, precision=hi)
        q = jnp.dot(stats_ref[1:2, :], m, precision=hi)
        den = den_ref[...]
        mean = s / den
        var = q / den - mean * mean
        scale = c_ref[...] * lax.rsqrt(var + EPS)
        bias = -mean * scale
        sb_ref[0:1, :] = lax.dot_general(
            scale, m, (((1,), (1,)), ((), ())), precision=hi
        )
        sb_ref[1:2, :] = lax.dot_general(
            bias, m, (((1,), (1,)), ((), ())), precision=hi
        )

    x = jnp.reshape(g_ref[...], (BB, PAD_D))
    xn = x * sb_ref[0:1, :] + sb_ref[1:2, :]
    out[...] = xn[:, 0:OUT_D]


def _tc_affine(stats, c_all, graw1):
    return pl.pallas_call(
        _affine_body,
        grid=(NBB,),
        in_specs=[
            pl.BlockSpec((8, PAD_D), lambda i: (0, 0)),
            pl.BlockSpec((PAD_D, NF), lambda i: (0, 0)),
            pl.BlockSpec((1, NF), lambda i: (0, 0)),
            pl.BlockSpec((1, NF), lambda i: (0, 0)),
            pl.BlockSpec((GBB, 128), lambda i: (i, 0)),
        ],
        out_specs=pl.BlockSpec((BB, OUT_D), lambda i: (i, 0)),
        out_shape=jax.ShapeDtypeStruct((B, OUT_D), jnp.float32),
        scratch_shapes=[pltpu.VMEM((8, PAD_D), jnp.float32)],
    )(stats, jnp.asarray(_M), c_all, jnp.asarray(_DEN), graw1)


def _linear8():
    return Layout(major_to_minor=(0, 1), tiling=((8,),))


@jax.jit
def kernel(input_ids, emb_0, emb_1, emb_2, fw_0, fw_1, fw_2, region_weights_raw):
    rw = jax.nn.softmax(region_weights_raw, axis=0)
    c_all = jnp.zeros((1, NF), jnp.float32)
    c_all = c_all.at[0, 0:F].set(fw_0[:, 0] * rw[0, 0])
    c_all = c_all.at[0, 32 : 32 + F].set(fw_1[:, 0] * rw[1, 0])
    c_all = c_all.at[0, 64 : 64 + F].set(fw_2[:, 0] * rw[2, 0])
    ids3 = input_ids.astype(jnp.int32).T.reshape(F, B // IDXROW, IDXROW)
    lin8, lin16, lin32 = _tc_linearize(emb_0.T, emb_1.T, emb_2.T)
    t8 = lin8.reshape(VOCAB, D8)
    t16 = lin16.reshape(VOCAB, D16)
    t32 = lin32.reshape(VOCAB, D32)
    graw = _make_sc_gather()(ids3, t8, t16, t32)
    graw1 = graw.reshape(GROWS, 128)
    stats = _tc_stats(jnp.asarray(_MASK), graw1)
    return _tc_affine(stats, c_all, graw1)


# speedup vs baseline: 1.4735x; 1.0010x over previous
import numpy as np

import jax
import jax.numpy as jnp
from jax import lax
from jax.experimental import pallas as pl
from jax.experimental.pallas import tpu as pltpu
from jax.experimental.pallas import tpu_sc as plsc

F = 26
B = 4096
EPS = 1e-5
NC, NS, L = 2, 16, 16
CH = 1024
NCH = B // CH
IDXROW = 128
GPC = CH // IDXROW

D8, D16, D32 = 8, 16, 32
COL16 = F * D8
COL32 = F * (D8 + D16)
OUT_D = F * (D8 + D16 + D32)
PAD_D = 1536
GROWS = B * PAD_D // 128
NBB = 16
BB = B // NBB
GBB = BB * PAD_D // 128
NF = 96

_M = np.zeros((PAD_D, NF), np.float32)
_DEN = np.ones((1, NF), np.float32)
_MASK = np.zeros((1, PAD_D), np.float32)
_MASK[0, :OUT_D] = 1.0
for _f in range(F):
    _M[_f * D8 : (_f + 1) * D8, _f] = 1.0
    _DEN[0, _f] = B * D8
    _M[COL16 + _f * D16 : COL16 + (_f + 1) * D16, 32 + _f] = 1.0
    _DEN[0, 32 + _f] = B * D16
    _M[COL32 + _f * D32 : COL32 + (_f + 1) * D32, 64 + _f] = 1.0
    _DEN[0, 64 + _f] = B * D32


def _sc_body(ids3, t8, t16, t32, graw, idxv, b8, b16, b32, sem):
    wid = lax.axis_index("s") * NC + lax.axis_index("c")

    def unit(table, f, d, buf, col_base):
        pltpu.sync_copy(ids3.at[f], idxv)
        col = col_base + f * d

        def chunk(c, carry):
            hs = [
                pltpu.async_copy(
                    table.at[idxv.at[c * GPC + j]],
                    buf.at[pl.ds(j * IDXROW, IDXROW)],
                    sem,
                )
                for j in range(GPC)
            ]
            for h in hs:
                h.wait()
            pltpu.sync_copy(buf, graw.at[pl.ds(c * CH, CH), pl.ds(col, d)])
            return carry

        lax.fori_loop(0, NCH, chunk, 0)

    @pl.when(wid < F)
    def _():
        unit(t32, wid, D32, b32, COL32)
        unit(t16, wid, D16, b16, COL16)

    @pl.when(wid >= F)
    def _():
        def d8_slot(k, carry):
            f = (wid - F) + (NC * NS - F) * k

            @pl.when(f < F)
            def _():
                unit(t8, f, D8, b8, 0)

            return carry

        lax.fori_loop(0, 5, d8_slot, 0)


def _make_sc_gather():
    return pl.kernel(
        _sc_body,
        out_type=jax.ShapeDtypeStruct((B, PAD_D), jnp.float32),
        mesh=plsc.VectorSubcoreMesh(
            core_axis_name="c", subcore_axis_name="s", num_cores=NC, num_subcores=NS
        ),
        scratch_types=[
            pltpu.VMEM((B // IDXROW, IDXROW), jnp.int32),
            pltpu.VMEM((CH, D8), jnp.float32),
            pltpu.VMEM((CH, D16), jnp.float32),
            pltpu.VMEM((CH, D32), jnp.float32),
            pltpu.SemaphoreType.DMA,
        ],
        compiler_params=pltpu.CompilerParams(use_tc_tiling_on_sc=False),
    )


VB = 8192
VOCAB = 100000
NVB = -(-VOCAB // VB)


def _lin_body(x8, x16, x32, y8, y16, y32):
    for x, y, d in ((x8, y8, D8), (x16, y16, D16), (x32, y32, D32)):
        g = 128 // d
        xt = x[...].T
        xg = jnp.reshape(xt, (VB // g, g, d))
        for k in range(g):
            y[:, k * d : (k + 1) * d] = xg[:, k, :]


def _tc_linearize(e0t, e1t, e2t):
    return pl.pallas_call(
        _lin_body,
        grid=(NVB,),
        in_specs=[
            pl.BlockSpec((D8, VB), lambda i: (0, i)),
            pl.BlockSpec((D16, VB), lambda i: (0, i)),
            pl.BlockSpec((D32, VB), lambda i: (0, i)),
        ],
        out_specs=[
            pl.BlockSpec((VB * D8 // 128, 128), lambda i: (i, 0)),
            pl.BlockSpec((VB * D16 // 128, 128), lambda i: (i, 0)),
            pl.BlockSpec((VB * D32 // 128, 128), lambda i: (i, 0)),
        ],
        out_shape=[
            jax.ShapeDtypeStruct((VOCAB * D8 // 128, 128), jnp.float32),
            jax.ShapeDtypeStruct((VOCAB * D16 // 128, 128), jnp.float32),
            jax.ShapeDtypeStruct((VOCAB * D32 // 128, 128), jnp.float32),
        ],
    )(e0t, e1t, e2t)


def _stats_body(mask_ref, g_ref, out):
    @pl.when(pl.program_id(0) == 0)
    def _():
        out[...] = jnp.zeros_like(out)

    x = jnp.reshape(g_ref[...], (BB, PAD_D))
    x = jnp.where(mask_ref[...] != 0.0, x, 0.0)
    out[0:1, :] += jnp.sum(x, axis=0, keepdims=True)
    out[1:2, :] += jnp.sum(x * x, axis=0, keepdims=True)


def _tc_stats(mask, graw1):
    return pl.pallas_call(
        _stats_body,
        grid=(NBB,),
        in_specs=[
            pl.BlockSpec((1, PAD_D), lambda i: (0, 0)),
            pl.BlockSpec((GBB, 128), lambda i: (i, 0)),
        ],
        out_specs=pl.BlockSpec((8, PAD_D), lambda i: (0, 0)),
        out_shape=jax.ShapeDtypeStruct((8, PAD_D), jnp.float32),
    )(mask, graw1)


def _affine_body(stats_ref, m_ref, c_ref, den_ref, g_ref, out, sb_ref):
    @pl.when(pl.program_id(0) == 0)
    def _():
        m = m_ref[...]
        hi = lax.Precision.HIGHEST
        s = jnp.dot(stats_ref[0:1, :], m, precision=hi)
        q = jnp.dot(stats_ref[1:2, :], m, precision=hi)
        den = den_ref[...]
        mean = s / den
        var = q / den - mean * mean
        scale = c_ref[...] * lax.rsqrt(var + EPS)
        bias = -mean * scale
        sb_ref[0:1, :] = lax.dot_general(
            scale, m, (((1,), (1,)), ((), ())), precision=hi
        )
        sb_ref[1:2, :] = lax.dot_general(
            bias, m, (((1,), (1,)), ((), ())), precision=hi
        )

    x = jnp.reshape(g_ref[...], (BB, PAD_D))
    xn = x * sb_ref[0:1, :] + sb_ref[1:2, :]
    out[...] = xn[:, 0:OUT_D]


def _tc_affine(stats, c_all, graw1):
    return pl.pallas_call(
        _affine_body,
        grid=(NBB,),
        in_specs=[
            pl.BlockSpec((8, PAD_D), lambda i: (0, 0)),
            pl.BlockSpec((PAD_D, NF), lambda i: (0, 0)),
            pl.BlockSpec((1, NF), lambda i: (0, 0)),
            pl.BlockSpec((1, NF), lambda i: (0, 0)),
            pl.BlockSpec((GBB, 128), lambda i: (i, 0)),
        ],
        out_specs=pl.BlockSpec((BB, OUT_D), lambda i: (i, 0)),
        out_shape=jax.ShapeDtypeStruct((B, OUT_D), jnp.float32),
        scratch_shapes=[pltpu.VMEM((8, PAD_D), jnp.float32)],
    )(stats, jnp.asarray(_M), c_all, jnp.asarray(_DEN), graw1)


@jax.jit
def kernel(input_ids, emb_0, emb_1, emb_2, fw_0, fw_1, fw_2, region_weights_raw):
    rw = jax.nn.softmax(region_weights_raw, axis=0)
    c_all = jnp.zeros((1, NF), jnp.float32)
    c_all = c_all.at[0, 0:F].set(fw_0[:, 0] * rw[0, 0])
    c_all = c_all.at[0, 32 : 32 + F].set(fw_1[:, 0] * rw[1, 0])
    c_all = c_all.at[0, 64 : 64 + F].set(fw_2[:, 0] * rw[2, 0])
    ids3 = input_ids.astype(jnp.int32).T.reshape(F, B // IDXROW, IDXROW)
    lin8, lin16, lin32 = _tc_linearize(emb_0.T, emb_1.T, emb_2.T)
    t8 = lin8.reshape(VOCAB, D8)
    t16 = lin16.reshape(VOCAB, D16)
    t32 = lin32.reshape(VOCAB, D32)
    graw = _make_sc_gather()(ids3, t8, t16, t32)
    graw1 = graw.reshape(GROWS, 128)
    stats = _tc_stats(jnp.asarray(_MASK), graw1)
    return _tc_affine(stats, c_all, graw1)
